# baseline (speedup 1.0000x reference)
import functools

import jax
import jax.numpy as jnp
from jax import lax
from jax.experimental import pallas as pl
from jax.experimental.pallas import tpu as pltpu
from jax.experimental.pallas import tpu_sc as plsc

_N = 10000
_E = 320000
_D = 128
_H = 64
_K = 10
_ALPHA = 0.1

_L = 16
_TILES = 16
_EPT = _E // _TILES
_EPTA = 20096
_EPTB = 19968
_BIGT = 4
_NP = 10240
_CHUNK = _NP // _TILES
_ZLAST = _N - (_TILES - 1) * _CHUNK


def _mlp_body(x_ref, w1_ref, b1_ref, w2_ref, o_ref):
    xb = x_ref[...]
    h = lax.dot_general(xb, w1_ref[...], (((1,), (1,)), ((), ())),
                        preferred_element_type=jnp.float32)
    h = jnp.maximum(h + b1_ref[...], 0.0)
    z = lax.dot_general(w2_ref[...], h, (((1,), (1,)), ((), ())),
                        preferred_element_type=jnp.float32)
    o_ref[...] = z[None]


_BZ = 1024


def _mlp_call(x, W1, b1r, W2):
    return pl.pallas_call(
        _mlp_body,
        grid=(_NP // _BZ,),
        in_specs=[
            pl.BlockSpec((_BZ, _D), lambda i: (i, 0)),
            pl.BlockSpec((_H, _D), lambda i: (0, 0)),
            pl.BlockSpec((1, _H), lambda i: (0, 0)),
            pl.BlockSpec((1, _H), lambda i: (0, 0)),
        ],
        out_specs=pl.BlockSpec((1, 1, _BZ), lambda i: (i, 0, 0)),
        out_shape=jax.ShapeDtypeStruct((_NP // _BZ, 1, _BZ), jnp.float32),
    )(x, W1, b1r, W2)


def _stage_edges(ei_h, esd, wid):
    @pl.when(wid < _BIGT)
    def _():
        pltpu.sync_copy(ei_h.at[:, pl.ds(wid * _EPTA, _EPTA)], esd)

    @pl.when(wid >= _BIGT)
    def _():
        pltpu.sync_copy(
            ei_h.at[:, pl.ds(_BIGT * _EPTA + (wid - _BIGT) * _EPTB, _EPTB)],
            esd.at[:, pl.ds(0, _EPTB)])
        pad16 = jnp.full((_L,), _NP - 1, jnp.int32)
        def fill(i, _):
            esd[0, pl.ds(_EPTB + i * _L, _L)] = pad16
            esd[1, pl.ds(_EPTB + i * _L, _L)] = pad16
            return 0
        lax.fori_loop(0, (_EPTA - _EPTB) // _L, fill, 0)


def _deg_body(ei_h, deg_h, esd, agg, red, degc, shp, sem):
    wid = lax.axis_index("s")
    zero16 = jnp.zeros((_L,), jnp.float32)
    ones16 = jnp.ones((_L,), jnp.float32)
    _stage_edges(ei_h, esd, wid)

    @plsc.parallel_loop(0, _NP // _L, unroll=8)
    def _(i):
        agg[pl.ds(i * _L, _L)] = zero16

    @plsc.parallel_loop(0, _EPTA // _L, unroll=8)
    def _(i):
        dv = esd[1, pl.ds(i * _L, _L)]
        plsc.addupdate_scatter(agg, [dv], ones16)

    pltpu.sync_copy(agg, shp.at[wid])
    plsc.subcore_barrier()
    pltpu.sync_copy(shp.at[:, pl.ds(wid * _CHUNK, _CHUNK)], red)

    def _fin(i, _):
        ds = pl.ds(i * _L, _L)
        acc = red[0, ds]
        for t in range(1, _TILES):
            acc = acc + red[t, ds]
        degc[ds] = acc + 1.0
        return 0
    lax.fori_loop(0, _CHUNK // _L, _fin, 0)
    pltpu.sync_copy(degc, deg_h.at[pl.ds(wid * _CHUNK, _CHUNK)])


def _deg_call(edge_index):
    mesh = plsc.VectorSubcoreMesh(core_axis_name="c", subcore_axis_name="s",
                                  num_cores=1, num_subcores=_TILES)
    return pl.kernel(
        _deg_body,
        out_type=jax.ShapeDtypeStruct((_NP,), jnp.float32),
        mesh=mesh,
        scratch_types=[
            pltpu.VMEM((2, _EPTA), jnp.int32),
            pltpu.VMEM((_NP,), jnp.float32),
            pltpu.VMEM((_TILES, _CHUNK), jnp.float32),
            pltpu.VMEM((_CHUNK,), jnp.float32),
            pltpu.VMEM_SHARED((_TILES, _NP), jnp.float32),
            pltpu.SemaphoreType.DMA,
        ],
        compiler_params=pltpu.CompilerParams(needs_layout_passes=False),
    )(edge_index)

def _rsqrt_newton(x):
    i = jnp.int32(0x5F3759DF) - (plsc.bitcast(x, jnp.int32) >> 1)
    r = plsc.bitcast(i, jnp.float32)
    for _ in range(3):
        r = r * (1.5 - 0.5 * x * r * r)
    return r


def _prop_body(ei_h, z0_h, deg_h, b2_h, out_h,
               esd, z0f, y, agg, red, y0c, dinvc, dsqc, outc, b2v,
               shp, shy, shz, sem):
    wid = lax.axis_index("s")
    zero16 = jnp.zeros((_L,), jnp.float32)
    last = _TILES - 1

    _stage_edges(ei_h, esd, wid)
    pltpu.sync_copy(b2_h, b2v)
    pltpu.sync_copy(deg_h.at[pl.ds(wid * _CHUNK, _CHUNK)], dsqc)
    pltpu.sync_copy(z0_h, z0f)

    def _zero_agg():
        @plsc.parallel_loop(0, _NP // _L, unroll=8)
        def _(i):
            agg[pl.ds(i * _L, _L)] = zero16

    def _scatter_round():
        @plsc.parallel_loop(0, _EPTA // _L, unroll=8)
        def _(i):
            sv = esd[0, pl.ds(i * _L, _L)]
            dv = esd[1, pl.ds(i * _L, _L)]
            vals = plsc.load_gather(y, [sv])
            plsc.addupdate_scatter(agg, [dv], vals)

    def _publish_and_reduce(shp):
        pltpu.sync_copy(agg, shp.at[wid])
        plsc.subcore_barrier()
        pltpu.sync_copy(shp.at[:, pl.ds(wid * _CHUNK, _CHUNK)], red)

    def _reduce16(i):
        ds = pl.ds(i * _L, _L)
        acc = red[0, ds]
        for t in range(1, _TILES):
            acc = acc + red[t, ds]
        return ds, acc

    def run(shp, shy, shz):
        def _zoutc(i, _):
            outc[pl.ds(i * _L, _L)] = zero16
            return 0
        lax.fori_loop(0, _CHUNK // _L, _zoutc, 0)
        pltpu.sync_copy(outc, shz.at[pl.ds(wid * _CHUNK, _CHUNK)])
        _zero_agg()

        iota16 = lax.iota(jnp.int32, _L)
        def _deg_fin(i, _):
            ds = pl.ds(i * _L, _L)
            deg = dsqc[ds]
            r = _rsqrt_newton(deg)
            dinvc[ds] = r * r
            dsqc[ds] = deg * r
            n = wid * _CHUNK + i * _L
            z = z0f[n >> 10, 0, pl.ds(n & (_BZ - 1), _L)]
            y0c[ds] = jnp.where(n + iota16 < _N, r * z, 0.0)
            return 0
        lax.fori_loop(0, _CHUNK // _L, _deg_fin, 0)
        pltpu.sync_copy(y0c, shy.at[pl.ds(wid * _CHUNK, _CHUNK)])
        plsc.subcore_barrier()
        pltpu.sync_copy(shy, y)

        def _round(k, _):
            _scatter_round()
            _publish_and_reduce(shp)
            zdma = pltpu.async_copy(shz, agg, sem)
            def _upd(i, _):
                ds, acc = _reduce16(i)
                yold = y[pl.ds(wid * _CHUNK + i * _L, _L)]
                outc[ds] = ((1.0 - _ALPHA) * dinvc[ds] * (acc + yold)
                            + _ALPHA * y0c[ds])
                return 0
            lax.fori_loop(0, _CHUNK // _L, _upd, 0)
            pltpu.sync_copy(outc, shy.at[pl.ds(wid * _CHUNK, _CHUNK)])
            plsc.subcore_barrier()
            pltpu.sync_copy(shy, y)
            zdma.wait()
            return 0
        lax.fori_loop(0, _K - 1, _round, 0)

        _scatter_round()
        _publish_and_reduce(shp)
        bv = b2v[...]
        def _upd_fin(i, _):
            ds, acc = _reduce16(i)
            yold = y[pl.ds(wid * _CHUNK + i * _L, _L)]
            ynew = ((1.0 - _ALPHA) * dinvc[ds] * (acc + yold)
                    + _ALPHA * y0c[ds])
            outc[ds] = dsqc[ds] * ynew + bv
            return 0
        lax.fori_loop(0, _CHUNK // _L, _upd_fin, 0)

        @pl.when(wid < last)
        def _():
            pltpu.sync_copy(outc, out_h.at[pl.ds(wid * _CHUNK, _CHUNK)])

        @pl.when(wid == last)
        def _():
            pltpu.sync_copy(outc.at[pl.ds(0, _ZLAST)],
                            out_h.at[pl.ds(last * _CHUNK, _ZLAST)])

    run(shp, shy, shz)


def _prop_call(edge_index, z0, deg, b2):
    mesh = plsc.VectorSubcoreMesh(core_axis_name="c", subcore_axis_name="s",
                                  num_cores=1, num_subcores=_TILES)
    return pl.kernel(
        _prop_body,
        out_type=jax.ShapeDtypeStruct((_N,), jnp.float32),
        mesh=mesh,
        scratch_types=[
            pltpu.VMEM((2, _EPTA), jnp.int32),
            pltpu.VMEM((_NP // _BZ, 1, _BZ), jnp.float32),
            pltpu.VMEM((_NP,), jnp.float32),
            pltpu.VMEM((_NP,), jnp.float32),
            pltpu.VMEM((_TILES, _CHUNK), jnp.float32),
            pltpu.VMEM((_CHUNK,), jnp.float32),
            pltpu.VMEM((_CHUNK,), jnp.float32),
            pltpu.VMEM((_CHUNK,), jnp.float32),
            pltpu.VMEM((_CHUNK,), jnp.float32),
            pltpu.VMEM((_L,), jnp.float32),
            pltpu.VMEM_SHARED((_TILES, _NP), jnp.float32),
            pltpu.VMEM_SHARED((_NP,), jnp.float32),
            pltpu.VMEM_SHARED((_NP,), jnp.float32),
            pltpu.SemaphoreType.DMA,
        ],
        compiler_params=pltpu.CompilerParams(needs_layout_passes=False),
    )(edge_index, z0, deg, jnp.broadcast_to(b2, (_L,)))


def kernel(x, edge_index, W1, b1, W2, b2):
    deg = _deg_call(edge_index)
    z0 = _mlp_call(x, W1, b1.reshape(1, _H), W2)
    outp = _prop_call(edge_index, z0, deg, b2)
    return outp.reshape(_N, 1)


# speedup vs baseline: 1.0161x; 1.0161x over previous
import functools

import jax
import jax.numpy as jnp
from jax import lax
from jax.experimental import pallas as pl
from jax.experimental.pallas import tpu as pltpu
from jax.experimental.pallas import tpu_sc as plsc

_N = 10000
_E = 320000
_D = 128
_H = 64
_K = 10
_ALPHA = 0.1

_L = 16
_TILES = 16
_EPT = _E // _TILES
_EPTA = 20096
_EPTB = 19968
_BIGT = 4
_NP = 10240
_CHUNK = _NP // _TILES
_ZLAST = _N - (_TILES - 1) * _CHUNK


def _mlp_body(x_ref, w1_ref, b1_ref, w2_ref, o_ref):
    xb = x_ref[...]
    h = lax.dot_general(xb, w1_ref[...], (((1,), (1,)), ((), ())),
                        preferred_element_type=jnp.float32)
    h = jnp.maximum(h + b1_ref[...], 0.0)
    z = lax.dot_general(w2_ref[...], h, (((1,), (1,)), ((), ())),
                        preferred_element_type=jnp.float32)
    o_ref[...] = z[None]


_BZ = 1024


def _mlp_call(x, W1, b1r, W2):
    return pl.pallas_call(
        _mlp_body,
        grid=(_NP // _BZ,),
        in_specs=[
            pl.BlockSpec((_BZ, _D), lambda i: (i, 0)),
            pl.BlockSpec((_H, _D), lambda i: (0, 0)),
            pl.BlockSpec((1, _H), lambda i: (0, 0)),
            pl.BlockSpec((1, _H), lambda i: (0, 0)),
        ],
        out_specs=pl.BlockSpec((1, 1, _BZ), lambda i: (i, 0, 0)),
        out_shape=jax.ShapeDtypeStruct((_NP // _BZ, 1, _BZ), jnp.float32),
    )(x, W1, b1r, W2)


def _stage_edges(ei_h, esd, wid):
    @pl.when(wid < _BIGT)
    def _():
        pltpu.sync_copy(ei_h.at[:, pl.ds(wid * _EPTA, _EPTA)], esd)

    @pl.when(wid >= _BIGT)
    def _():
        pltpu.sync_copy(
            ei_h.at[:, pl.ds(_BIGT * _EPTA + (wid - _BIGT) * _EPTB, _EPTB)],
            esd.at[:, pl.ds(0, _EPTB)])
        pad16 = jnp.full((_L,), _NP - 1, jnp.int32)
        def fill(i, _):
            esd[0, pl.ds(_EPTB + i * _L, _L)] = pad16
            esd[1, pl.ds(_EPTB + i * _L, _L)] = pad16
            return 0
        lax.fori_loop(0, (_EPTA - _EPTB) // _L, fill, 0)


def _deg_body(ei_h, deg_h, esd, agg, red, degc, shp, sem):
    wid = lax.axis_index("s")
    zero16 = jnp.zeros((_L,), jnp.float32)
    ones16 = jnp.ones((_L,), jnp.float32)
    _stage_edges(ei_h, esd, wid)

    @plsc.parallel_loop(0, _NP // _L, unroll=8)
    def _(i):
        agg[pl.ds(i * _L, _L)] = zero16

    @plsc.parallel_loop(0, _EPTA // _L, unroll=8)
    def _(i):
        dv = esd[1, pl.ds(i * _L, _L)]
        plsc.addupdate_scatter(agg, [dv], ones16)

    pltpu.sync_copy(agg, shp.at[wid])
    plsc.subcore_barrier()
    pltpu.sync_copy(shp.at[:, pl.ds(wid * _CHUNK, _CHUNK)], red)

    def _fin(i, _):
        ds = pl.ds(i * _L, _L)
        acc = red[0, ds]
        for t in range(1, _TILES):
            acc = acc + red[t, ds]
        degc[ds] = acc + 1.0
        return 0
    lax.fori_loop(0, _CHUNK // _L, _fin, 0)
    pltpu.sync_copy(degc, deg_h.at[pl.ds(wid * _CHUNK, _CHUNK)])


def _deg_call(edge_index):
    mesh = plsc.VectorSubcoreMesh(core_axis_name="c", subcore_axis_name="s",
                                  num_cores=1, num_subcores=_TILES)
    return pl.kernel(
        _deg_body,
        out_type=jax.ShapeDtypeStruct((_NP,), jnp.float32),
        mesh=mesh,
        scratch_types=[
            pltpu.VMEM((2, _EPTA), jnp.int32),
            pltpu.VMEM((_NP,), jnp.float32),
            pltpu.VMEM((_TILES, _CHUNK), jnp.float32),
            pltpu.VMEM((_CHUNK,), jnp.float32),
            pltpu.VMEM_SHARED((_TILES, _NP), jnp.float32),
            pltpu.SemaphoreType.DMA,
        ],
        compiler_params=pltpu.CompilerParams(needs_layout_passes=False),
    )(edge_index)

def _rsqrt_newton(x):
    i = jnp.int32(0x5F3759DF) - (plsc.bitcast(x, jnp.int32) >> 1)
    r = plsc.bitcast(i, jnp.float32)
    for _ in range(3):
        r = r * (1.5 - 0.5 * x * r * r)
    return r


def _prop_body(ei_h, z0_h, deg_h, b2_h, out_h,
               esd, z0f, y, agg, red, y0c, dinvc, dsqc, outc, b2v,
               shp, shy, shz, sem):
    wid = lax.axis_index("s")
    zero16 = jnp.zeros((_L,), jnp.float32)
    last = _TILES - 1

    hb = pltpu.async_copy(b2_h, b2v, sem)
    hz = pltpu.async_copy(z0_h, z0f, sem)
    hd = pltpu.async_copy(deg_h.at[pl.ds(wid * _CHUNK, _CHUNK)], dsqc, sem)
    _stage_edges(ei_h, esd, wid)

    def _zero_agg():
        @plsc.parallel_loop(0, _NP // _L, unroll=8)
        def _(i):
            agg[pl.ds(i * _L, _L)] = zero16

    def _scatter_round():
        @plsc.parallel_loop(0, _EPTA // _L, unroll=16)
        def _(i):
            sv = esd[0, pl.ds(i * _L, _L)]
            dv = esd[1, pl.ds(i * _L, _L)]
            vals = plsc.load_gather(y, [sv])
            plsc.addupdate_scatter(agg, [dv], vals)

    def _publish_and_reduce(shp):
        pltpu.sync_copy(agg, shp.at[wid])
        plsc.subcore_barrier()
        pltpu.sync_copy(shp.at[:, pl.ds(wid * _CHUNK, _CHUNK)], red)

    def _reduce16(i):
        ds = pl.ds(i * _L, _L)
        acc = red[0, ds]
        for t in range(1, _TILES):
            acc = acc + red[t, ds]
        return ds, acc

    def run(shp, shy, shz):
        def _zoutc(i, _):
            outc[pl.ds(i * _L, _L)] = zero16
            return 0
        lax.fori_loop(0, _CHUNK // _L, _zoutc, 0)
        pltpu.sync_copy(outc, shz.at[pl.ds(wid * _CHUNK, _CHUNK)])
        _zero_agg()
        hb.wait()
        hz.wait()
        hd.wait()

        iota16 = lax.iota(jnp.int32, _L)
        def _deg_fin(i, _):
            ds = pl.ds(i * _L, _L)
            deg = dsqc[ds]
            r = _rsqrt_newton(deg)
            dinvc[ds] = r * r
            dsqc[ds] = deg * r
            n = wid * _CHUNK + i * _L
            z = z0f[n >> 10, 0, pl.ds(n & (_BZ - 1), _L)]
            y0c[ds] = jnp.where(n + iota16 < _N, r * z, 0.0)
            return 0
        lax.fori_loop(0, _CHUNK // _L, _deg_fin, 0)
        pltpu.sync_copy(y0c, shy.at[pl.ds(wid * _CHUNK, _CHUNK)])
        plsc.subcore_barrier()
        pltpu.sync_copy(shy, y)

        def _round(k, _):
            _scatter_round()
            _publish_and_reduce(shp)
            zdma = pltpu.async_copy(shz, agg, sem)
            def _upd(i, _):
                ds, acc = _reduce16(i)
                yold = y[pl.ds(wid * _CHUNK + i * _L, _L)]
                outc[ds] = ((1.0 - _ALPHA) * dinvc[ds] * (acc + yold)
                            + _ALPHA * y0c[ds])
                return 0
            lax.fori_loop(0, _CHUNK // _L, _upd, 0)
            pltpu.sync_copy(outc, shy.at[pl.ds(wid * _CHUNK, _CHUNK)])
            plsc.subcore_barrier()
            pltpu.sync_copy(shy, y)
            zdma.wait()
            return 0
        lax.fori_loop(0, _K - 1, _round, 0)

        _scatter_round()
        _publish_and_reduce(shp)
        bv = b2v[...]
        def _upd_fin(i, _):
            ds, acc = _reduce16(i)
            yold = y[pl.ds(wid * _CHUNK + i * _L, _L)]
            ynew = ((1.0 - _ALPHA) * dinvc[ds] * (acc + yold)
                    + _ALPHA * y0c[ds])
            outc[ds] = dsqc[ds] * ynew + bv
            return 0
        lax.fori_loop(0, _CHUNK // _L, _upd_fin, 0)

        @pl.when(wid < last)
        def _():
            pltpu.sync_copy(outc, out_h.at[pl.ds(wid * _CHUNK, _CHUNK)])

        @pl.when(wid == last)
        def _():
            pltpu.sync_copy(outc.at[pl.ds(0, _ZLAST)],
                            out_h.at[pl.ds(last * _CHUNK, _ZLAST)])

    run(shp, shy, shz)


def _prop_call(edge_index, z0, deg, b2):
    mesh = plsc.VectorSubcoreMesh(core_axis_name="c", subcore_axis_name="s",
                                  num_cores=1, num_subcores=_TILES)
    return pl.kernel(
        _prop_body,
        out_type=jax.ShapeDtypeStruct((_N,), jnp.float32),
        mesh=mesh,
        scratch_types=[
            pltpu.VMEM((2, _EPTA), jnp.int32),
            pltpu.VMEM((_NP // _BZ, 1, _BZ), jnp.float32),
            pltpu.VMEM((_NP,), jnp.float32),
            pltpu.VMEM((_NP,), jnp.float32),
            pltpu.VMEM((_TILES, _CHUNK), jnp.float32),
            pltpu.VMEM((_CHUNK,), jnp.float32),
            pltpu.VMEM((_CHUNK,), jnp.float32),
            pltpu.VMEM((_CHUNK,), jnp.float32),
            pltpu.VMEM((_CHUNK,), jnp.float32),
            pltpu.VMEM((_L,), jnp.float32),
            pltpu.VMEM_SHARED((_TILES, _NP), jnp.float32),
            pltpu.VMEM_SHARED((_NP,), jnp.float32),
            pltpu.VMEM_SHARED((_NP,), jnp.float32),
            pltpu.SemaphoreType.DMA,
        ],
        compiler_params=pltpu.CompilerParams(needs_layout_passes=False),
    )(edge_index, z0, deg, jnp.broadcast_to(b2, (_L,)))


def kernel(x, edge_index, W1, b1, W2, b2):
    deg = _deg_call(edge_index)
    z0 = _mlp_call(x, W1, b1.reshape(1, _H), W2)
    outp = _prop_call(edge_index, z0, deg, b2)
    return outp.reshape(_N, 1)


# speedup vs baseline: 1.0182x; 1.0021x over previous
import jax
import jax.numpy as jnp
from jax import lax
from jax.experimental import pallas as pl
from jax.experimental.pallas import tpu as pltpu
from jax.experimental.pallas import tpu_sc as plsc

_N = 10000
_E = 320000
_D = 128
_H = 64
_K = 10
_ALPHA = 0.1

_L = 16
_TILES = 16
_EPTA = 20096
_EPTB = 19968
_BIGT = 4
_NP = 10240
_CHUNK = _NP // _TILES
_ZLAST = _N - (_TILES - 1) * _CHUNK


def _mlp_body(x_ref, w1_ref, b1_ref, w2_ref, o_ref):
    xb = x_ref[...]
    h = lax.dot_general(xb, w1_ref[...], (((1,), (1,)), ((), ())),
                        preferred_element_type=jnp.float32)
    h = jnp.maximum(h + b1_ref[...], 0.0)
    z = lax.dot_general(w2_ref[...], h, (((1,), (1,)), ((), ())),
                        preferred_element_type=jnp.float32)
    o_ref[...] = z[None]


_BZ = 1024


def _mlp_call(x, W1, b1r, W2):
    return pl.pallas_call(
        _mlp_body,
        grid=(_NP // _BZ,),
        in_specs=[
            pl.BlockSpec((_BZ, _D), lambda i: (i, 0)),
            pl.BlockSpec((_H, _D), lambda i: (0, 0)),
            pl.BlockSpec((1, _H), lambda i: (0, 0)),
            pl.BlockSpec((1, _H), lambda i: (0, 0)),
        ],
        out_specs=pl.BlockSpec((1, 1, _BZ), lambda i: (i, 0, 0)),
        out_shape=jax.ShapeDtypeStruct((_NP // _BZ, 1, _BZ), jnp.float32),
    )(x, W1, b1r, W2)


def _stage_edges(ei_h, esd, wid):
    @pl.when(wid < _BIGT)
    def _():
        pltpu.sync_copy(ei_h.at[:, pl.ds(wid * _EPTA, _EPTA)], esd)

    @pl.when(wid >= _BIGT)
    def _():
        pltpu.sync_copy(
            ei_h.at[:, pl.ds(_BIGT * _EPTA + (wid - _BIGT) * _EPTB, _EPTB)],
            esd.at[:, pl.ds(0, _EPTB)])
        pad16 = jnp.full((_L,), _NP - 1, jnp.int32)
        def fill(i, _):
            esd[0, pl.ds(_EPTB + i * _L, _L)] = pad16
            esd[1, pl.ds(_EPTB + i * _L, _L)] = pad16
            return 0
        lax.fori_loop(0, (_EPTA - _EPTB) // _L, fill, 0)


def _deg_body(ei_h, deg_h, esd, agg, red, degc, shp, sem):
    wid = lax.axis_index("s")
    zero16 = jnp.zeros((_L,), jnp.float32)
    ones16 = jnp.ones((_L,), jnp.float32)
    _stage_edges(ei_h, esd, wid)

    @plsc.parallel_loop(0, _NP // _L, unroll=8)
    def _(i):
        agg[pl.ds(i * _L, _L)] = zero16

    @plsc.parallel_loop(0, _EPTA // _L, unroll=8)
    def _(i):
        dv = esd[1, pl.ds(i * _L, _L)]
        plsc.addupdate_scatter(agg, [dv], ones16)

    pltpu.sync_copy(agg, shp.at[wid])
    plsc.subcore_barrier()
    pltpu.sync_copy(shp.at[:, pl.ds(wid * _CHUNK, _CHUNK)], red)

    def _fin(i, _):
        ds = pl.ds(i * _L, _L)
        acc = red[0, ds]
        for t in range(1, _TILES):
            acc = acc + red[t, ds]
        degc[ds] = acc + 1.0
        return 0
    lax.fori_loop(0, _CHUNK // _L, _fin, 0)
    pltpu.sync_copy(degc, deg_h.at[pl.ds(wid * _CHUNK, _CHUNK)])


def _deg_call(edge_index):
    mesh = plsc.VectorSubcoreMesh(core_axis_name="c", subcore_axis_name="s",
                                  num_cores=1, num_subcores=_TILES)
    return pl.kernel(
        _deg_body,
        out_type=jax.ShapeDtypeStruct((_NP,), jnp.float32),
        mesh=mesh,
        scratch_types=[
            pltpu.VMEM((2, _EPTA), jnp.int32),
            pltpu.VMEM((_NP,), jnp.float32),
            pltpu.VMEM((_TILES, _CHUNK), jnp.float32),
            pltpu.VMEM((_CHUNK,), jnp.float32),
            pltpu.VMEM_SHARED((_TILES, _NP), jnp.float32),
            pltpu.SemaphoreType.DMA,
        ],
        compiler_params=pltpu.CompilerParams(needs_layout_passes=False),
    )(edge_index)

def _rsqrt_newton(x):
    i = jnp.int32(0x5F3759DF) - (plsc.bitcast(x, jnp.int32) >> 1)
    r = plsc.bitcast(i, jnp.float32)
    for _ in range(3):
        r = r * (1.5 - 0.5 * x * r * r)
    return r


def _prop_body(ei_h, z0_h, deg_h, b2_h, out_h,
               esd, z0f, y, agg, red, y0c, dinvc, dsqc, outc, b2v,
               shp, shy, shz, sem):
    wid = lax.axis_index("s")
    zero16 = jnp.zeros((_L,), jnp.float32)
    last = _TILES - 1

    hb = pltpu.async_copy(b2_h, b2v, sem)
    hz = pltpu.async_copy(z0_h, z0f, sem)
    hd = pltpu.async_copy(deg_h.at[pl.ds(wid * _CHUNK, _CHUNK)], dsqc, sem)
    _stage_edges(ei_h, esd, wid)

    def _zero_agg():
        @plsc.parallel_loop(0, _NP // _L, unroll=8)
        def _(i):
            agg[pl.ds(i * _L, _L)] = zero16

    def _scatter_round():
        @plsc.parallel_loop(0, _EPTA // _L, unroll=16)
        def _(i):
            sv = esd[0, pl.ds(i * _L, _L)]
            dv = esd[1, pl.ds(i * _L, _L)]
            vals = plsc.load_gather(y, [sv])
            plsc.addupdate_scatter(agg, [dv], vals)

    def _publish_and_reduce(shp):
        pltpu.sync_copy(agg, shp.at[wid])
        plsc.subcore_barrier()
        pltpu.sync_copy(shp.at[:, pl.ds(wid * _CHUNK, _CHUNK)], red)

    def _reduce16(i):
        ds = pl.ds(i * _L, _L)
        acc = red[0, ds]
        for t in range(1, _TILES):
            acc = acc + red[t, ds]
        return ds, acc

    def run(shp, shy, shz):
        def _zoutc(i, _):
            outc[pl.ds(i * _L, _L)] = zero16
            return 0
        lax.fori_loop(0, _CHUNK // _L, _zoutc, 0)
        pltpu.sync_copy(outc, shz.at[pl.ds(wid * _CHUNK, _CHUNK)])
        _zero_agg()
        hb.wait()
        hz.wait()
        hd.wait()

        iota16 = lax.iota(jnp.int32, _L)
        def _deg_fin(i, _):
            ds = pl.ds(i * _L, _L)
            deg = dsqc[ds]
            r = _rsqrt_newton(deg)
            dinvc[ds] = r * r
            dsqc[ds] = deg * r
            n = wid * _CHUNK + i * _L
            z = z0f[n >> 10, 0, pl.ds(n & (_BZ - 1), _L)]
            y0c[ds] = jnp.where(n + iota16 < _N, r * z, 0.0)
            return 0
        lax.fori_loop(0, _CHUNK // _L, _deg_fin, 0)
        pltpu.sync_copy(y0c, shy.at[pl.ds(wid * _CHUNK, _CHUNK)])
        plsc.subcore_barrier()
        pltpu.sync_copy(shy, y)

        def _round(k, _):
            _scatter_round()
            _publish_and_reduce(shp)
            zdma = pltpu.async_copy(shz, agg, sem)
            def _upd(i, _):
                ds, acc = _reduce16(i)
                yold = y[pl.ds(wid * _CHUNK + i * _L, _L)]
                outc[ds] = ((1.0 - _ALPHA) * dinvc[ds] * (acc + yold)
                            + _ALPHA * y0c[ds])
                return 0
            lax.fori_loop(0, _CHUNK // _L, _upd, 0)
            pltpu.sync_copy(outc, shy.at[pl.ds(wid * _CHUNK, _CHUNK)])
            plsc.subcore_barrier()
            pltpu.sync_copy(shy, y)
            zdma.wait()
            return 0
        lax.fori_loop(0, _K - 1, _round, 0)

        _scatter_round()
        _publish_and_reduce(shp)
        bv = b2v[...]
        def _upd_fin(i, _):
            ds, acc = _reduce16(i)
            yold = y[pl.ds(wid * _CHUNK + i * _L, _L)]
            ynew = ((1.0 - _ALPHA) * dinvc[ds] * (acc + yold)
                    + _ALPHA * y0c[ds])
            outc[ds] = dsqc[ds] * ynew + bv
            return 0
        lax.fori_loop(0, _CHUNK // _L, _upd_fin, 0)

        @pl.when(wid < last)
        def _():
            pltpu.sync_copy(outc, out_h.at[pl.ds(wid * _CHUNK, _CHUNK)])

        @pl.when(wid == last)
        def _():
            pltpu.sync_copy(outc.at[pl.ds(0, _ZLAST)],
                            out_h.at[pl.ds(last * _CHUNK, _ZLAST)])

    run(shp, shy, shz)


def _prop_call(edge_index, z0, deg, b2):
    mesh = plsc.VectorSubcoreMesh(core_axis_name="c", subcore_axis_name="s",
                                  num_cores=1, num_subcores=_TILES)
    return pl.kernel(
        _prop_body,
        out_type=jax.ShapeDtypeStruct((_N,), jnp.float32),
        mesh=mesh,
        scratch_types=[
            pltpu.VMEM((2, _EPTA), jnp.int32),
            pltpu.VMEM((_NP // _BZ, 1, _BZ), jnp.float32),
            pltpu.VMEM((_NP,), jnp.float32),
            pltpu.VMEM((_NP,), jnp.float32),
            pltpu.VMEM((_TILES, _CHUNK), jnp.float32),
            pltpu.VMEM((_CHUNK,), jnp.float32),
            pltpu.VMEM((_CHUNK,), jnp.float32),
            pltpu.VMEM((_CHUNK,), jnp.float32),
            pltpu.VMEM((_CHUNK,), jnp.float32),
            pltpu.VMEM((_L,), jnp.float32),
            pltpu.VMEM_SHARED((_TILES, _NP), jnp.float32),
            pltpu.VMEM_SHARED((_NP,), jnp.float32),
            pltpu.VMEM_SHARED((_NP,), jnp.float32),
            pltpu.SemaphoreType.DMA,
        ],
        compiler_params=pltpu.CompilerParams(needs_layout_passes=False),
    )(edge_index, z0, deg, jnp.broadcast_to(b2, (_L,)))


def kernel(x, edge_index, W1, b1, W2, b2):
    deg = _deg_call(edge_index)
    z0 = _mlp_call(x, W1, b1.reshape(1, _H), W2)
    outp = _prop_call(edge_index, z0, deg, b2)
    return outp.reshape(_N, 1)


# speedup vs baseline: 1.0754x; 1.0562x over previous
import jax
import jax.numpy as jnp
from jax import lax
from jax.experimental import pallas as pl
from jax.experimental.pallas import tpu as pltpu
from jax.experimental.pallas import tpu_sc as plsc

_N = 10000
_E = 320000
_D = 128
_H = 64
_K = 10
_ALPHA = 0.1

_L = 16
_TILES = 16
_EPTA = 20096
_EPTB = 19968
_BIGT = 4
_NP = 10240
_CHUNK = _NP // _TILES
_ZLAST = _N - (_TILES - 1) * _CHUNK


def _mlp_body(x_ref, w1_ref, b1_ref, w2_ref, o_ref):
    xb = x_ref[...]
    h = lax.dot_general(xb, w1_ref[...], (((1,), (1,)), ((), ())),
                        preferred_element_type=jnp.float32)
    h = jnp.maximum(h + b1_ref[...], 0.0)
    z = lax.dot_general(w2_ref[...], h, (((1,), (1,)), ((), ())),
                        preferred_element_type=jnp.float32)
    o_ref[...] = z[None]


_BZ = 1024


def _mlp_call(x, W1, b1r, W2):
    return pl.pallas_call(
        _mlp_body,
        grid=(_NP // _BZ,),
        in_specs=[
            pl.BlockSpec((_BZ, _D), lambda i: (i, 0)),
            pl.BlockSpec((_H, _D), lambda i: (0, 0)),
            pl.BlockSpec((1, _H), lambda i: (0, 0)),
            pl.BlockSpec((1, _H), lambda i: (0, 0)),
        ],
        out_specs=pl.BlockSpec((1, 1, _BZ), lambda i: (i, 0, 0)),
        out_shape=jax.ShapeDtypeStruct((_NP // _BZ, 1, _BZ), jnp.float32),
    )(x, W1, b1r, W2)


def _stage_edges(ei_h, esd, wid):
    @pl.when(wid < _BIGT)
    def _():
        pltpu.sync_copy(ei_h.at[:, pl.ds(wid * _EPTA, _EPTA)], esd)

    @pl.when(wid >= _BIGT)
    def _():
        pltpu.sync_copy(
            ei_h.at[:, pl.ds(_BIGT * _EPTA + (wid - _BIGT) * _EPTB, _EPTB)],
            esd.at[:, pl.ds(0, _EPTB)])
        pad16 = jnp.full((_L,), _NP - 1, jnp.int32)
        def fill(i, _):
            esd[0, pl.ds(_EPTB + i * _L, _L)] = pad16
            esd[1, pl.ds(_EPTB + i * _L, _L)] = pad16
            return 0
        lax.fori_loop(0, (_EPTA - _EPTB) // _L, fill, 0)


def _deg_body(ei_h, deg_h, esd, agg, red, degc, shp, sem):
    wid = lax.axis_index("s")
    zero16 = jnp.zeros((_L,), jnp.float32)
    ones16 = jnp.ones((_L,), jnp.float32)
    _stage_edges(ei_h, esd, wid)

    @plsc.parallel_loop(0, _NP // _L, unroll=8)
    def _(i):
        agg[pl.ds(i * _L, _L)] = zero16

    @plsc.parallel_loop(0, _EPTA // _L, unroll=8)
    def _(i):
        dv = esd[1, pl.ds(i * _L, _L)]
        plsc.addupdate_scatter(agg, [dv], ones16)

    pltpu.sync_copy(agg, shp.at[wid])
    plsc.subcore_barrier()
    pltpu.sync_copy(shp.at[:, pl.ds(wid * _CHUNK, _CHUNK)], red)

    @plsc.parallel_loop(0, _CHUNK // _L, unroll=2)
    def _(i):
        ds = pl.ds(i * _L, _L)
        acc = red[0, ds]
        for t in range(1, _TILES):
            acc = acc + red[t, ds]
        degc[ds] = acc + 1.0
    pltpu.sync_copy(degc, deg_h.at[pl.ds(wid * _CHUNK, _CHUNK)])


def _deg_call(edge_index):
    mesh = plsc.VectorSubcoreMesh(core_axis_name="c", subcore_axis_name="s",
                                  num_cores=1, num_subcores=_TILES)
    return pl.kernel(
        _deg_body,
        out_type=jax.ShapeDtypeStruct((_NP,), jnp.float32),
        mesh=mesh,
        scratch_types=[
            pltpu.VMEM((2, _EPTA), jnp.int32),
            pltpu.VMEM((_NP,), jnp.float32),
            pltpu.VMEM((_TILES, _CHUNK), jnp.float32),
            pltpu.VMEM((_CHUNK,), jnp.float32),
            pltpu.VMEM_SHARED((_TILES, _NP), jnp.float32),
            pltpu.SemaphoreType.DMA,
        ],
        compiler_params=pltpu.CompilerParams(needs_layout_passes=False),
    )(edge_index)

def _rsqrt_newton(x):
    i = jnp.int32(0x5F3759DF) - (plsc.bitcast(x, jnp.int32) >> 1)
    r = plsc.bitcast(i, jnp.float32)
    for _ in range(3):
        r = r * (1.5 - 0.5 * x * r * r)
    return r


def _prop_body(ei_h, z0_h, deg_h, b2_h, out_h,
               esd, z0f, y, agg, red, y0c, dinvc, dsqc, outc, b2v,
               shp, shy, shz, sem):
    wid = lax.axis_index("s")
    zero16 = jnp.zeros((_L,), jnp.float32)
    last = _TILES - 1

    hb = pltpu.async_copy(b2_h, b2v, sem)
    hz = pltpu.async_copy(z0_h, z0f, sem)
    hd = pltpu.async_copy(deg_h.at[pl.ds(wid * _CHUNK, _CHUNK)], dsqc, sem)
    _stage_edges(ei_h, esd, wid)

    def _zero_agg():
        @plsc.parallel_loop(0, _NP // _L, unroll=8)
        def _(i):
            agg[pl.ds(i * _L, _L)] = zero16

    def _scatter_round():
        @plsc.parallel_loop(0, _EPTA // _L, unroll=16)
        def _(i):
            sv = esd[0, pl.ds(i * _L, _L)]
            dv = esd[1, pl.ds(i * _L, _L)]
            vals = plsc.load_gather(y, [sv])
            plsc.addupdate_scatter(agg, [dv], vals)

    def _publish_and_reduce(shp):
        pltpu.sync_copy(agg, shp.at[wid])
        plsc.subcore_barrier()
        pltpu.sync_copy(shp.at[:, pl.ds(wid * _CHUNK, _CHUNK)], red)

    def _reduce16(i):
        ds = pl.ds(i * _L, _L)
        acc = red[0, ds]
        for t in range(1, _TILES):
            acc = acc + red[t, ds]
        return ds, acc

    def run(shp, shy, shz):
        def _zoutc(i, _):
            outc[pl.ds(i * _L, _L)] = zero16
            return 0
        lax.fori_loop(0, _CHUNK // _L, _zoutc, 0)
        pltpu.sync_copy(outc, shz.at[pl.ds(wid * _CHUNK, _CHUNK)])
        _zero_agg()
        hb.wait()
        hz.wait()
        hd.wait()

        iota16 = lax.iota(jnp.int32, _L)
        def _deg_fin(i, _):
            ds = pl.ds(i * _L, _L)
            deg = dsqc[ds]
            r = _rsqrt_newton(deg)
            dinvc[ds] = r * r
            dsqc[ds] = deg * r
            n = wid * _CHUNK + i * _L
            z = z0f[n >> 10, 0, pl.ds(n & (_BZ - 1), _L)]
            y0c[ds] = jnp.where(n + iota16 < _N, r * z, 0.0)
            return 0
        lax.fori_loop(0, _CHUNK // _L, _deg_fin, 0)
        pltpu.sync_copy(y0c, shy.at[pl.ds(wid * _CHUNK, _CHUNK)])
        plsc.subcore_barrier()
        pltpu.sync_copy(shy, y)

        def _round(k, _):
            _scatter_round()
            _publish_and_reduce(shp)
            zdma = pltpu.async_copy(shz, agg, sem)

            @plsc.parallel_loop(0, _CHUNK // _L, unroll=2)
            def _(i):
                ds, acc = _reduce16(i)
                yold = y[pl.ds(wid * _CHUNK + i * _L, _L)]
                outc[ds] = ((1.0 - _ALPHA) * dinvc[ds] * (acc + yold)
                            + _ALPHA * y0c[ds])
            pltpu.sync_copy(outc, shy.at[pl.ds(wid * _CHUNK, _CHUNK)])
            plsc.subcore_barrier()
            pltpu.sync_copy(shy, y)
            zdma.wait()
            return 0
        lax.fori_loop(0, _K - 1, _round, 0)

        _scatter_round()
        _publish_and_reduce(shp)
        bv = b2v[...]

        @plsc.parallel_loop(0, _CHUNK // _L, unroll=2)
        def _(i):
            ds, acc = _reduce16(i)
            yold = y[pl.ds(wid * _CHUNK + i * _L, _L)]
            ynew = ((1.0 - _ALPHA) * dinvc[ds] * (acc + yold)
                    + _ALPHA * y0c[ds])
            outc[ds] = dsqc[ds] * ynew + bv

        @pl.when(wid < last)
        def _():
            pltpu.sync_copy(outc, out_h.at[pl.ds(wid * _CHUNK, _CHUNK)])

        @pl.when(wid == last)
        def _():
            pltpu.sync_copy(outc.at[pl.ds(0, _ZLAST)],
                            out_h.at[pl.ds(last * _CHUNK, _ZLAST)])

    run(shp, shy, shz)


def _prop_call(edge_index, z0, deg, b2):
    mesh = plsc.VectorSubcoreMesh(core_axis_name="c", subcore_axis_name="s",
                                  num_cores=1, num_subcores=_TILES)
    return pl.kernel(
        _prop_body,
        out_type=jax.ShapeDtypeStruct((_N,), jnp.float32),
        mesh=mesh,
        scratch_types=[
            pltpu.VMEM((2, _EPTA), jnp.int32),
            pltpu.VMEM((_NP // _BZ, 1, _BZ), jnp.float32),
            pltpu.VMEM((_NP,), jnp.float32),
            pltpu.VMEM((_NP,), jnp.float32),
            pltpu.VMEM((_TILES, _CHUNK), jnp.float32),
            pltpu.VMEM((_CHUNK,), jnp.float32),
            pltpu.VMEM((_CHUNK,), jnp.float32),
            pltpu.VMEM((_CHUNK,), jnp.float32),
            pltpu.VMEM((_CHUNK,), jnp.float32),
            pltpu.VMEM((_L,), jnp.float32),
            pltpu.VMEM_SHARED((_TILES, _NP), jnp.float32),
            pltpu.VMEM_SHARED((_NP,), jnp.float32),
            pltpu.VMEM_SHARED((_NP,), jnp.float32),
            pltpu.SemaphoreType.DMA,
        ],
        compiler_params=pltpu.CompilerParams(needs_layout_passes=False),
    )(edge_index, z0, deg, jnp.broadcast_to(b2, (_L,)))


def kernel(x, edge_index, W1, b1, W2, b2):
    deg = _deg_call(edge_index)
    z0 = _mlp_call(x, W1, b1.reshape(1, _H), W2)
    outp = _prop_call(edge_index, z0, deg, b2)
    return outp.reshape(_N, 1)


# speedup vs baseline: 1.0825x; 1.0066x over previous
import jax
import jax.numpy as jnp
from jax import lax
from jax.experimental import pallas as pl
from jax.experimental.pallas import tpu as pltpu
from jax.experimental.pallas import tpu_sc as plsc

_N = 10000
_E = 320000
_D = 128
_H = 64
_K = 10
_ALPHA = 0.1

_L = 16
_TILES = 16
_EPTA = 20096
_EPTB = 19968
_BIGT = 4
_NP = 10240
_CHUNK = _NP // _TILES
_ZLAST = _N - (_TILES - 1) * _CHUNK


def _mlp_body(x_ref, w1_ref, b1_ref, w2_ref, o_ref):
    xb = x_ref[...]
    h = lax.dot_general(xb, w1_ref[...], (((1,), (1,)), ((), ())),
                        preferred_element_type=jnp.float32)
    h = jnp.maximum(h + b1_ref[...], 0.0)
    z = lax.dot_general(w2_ref[...], h, (((1,), (1,)), ((), ())),
                        preferred_element_type=jnp.float32)
    o_ref[...] = z[None]


_BZ = 1024


def _mlp_call(x, W1, b1r, W2):
    return pl.pallas_call(
        _mlp_body,
        grid=(_NP // _BZ,),
        in_specs=[
            pl.BlockSpec((_BZ, _D), lambda i: (i, 0)),
            pl.BlockSpec((_H, _D), lambda i: (0, 0)),
            pl.BlockSpec((1, _H), lambda i: (0, 0)),
            pl.BlockSpec((1, _H), lambda i: (0, 0)),
        ],
        out_specs=pl.BlockSpec((1, 1, _BZ), lambda i: (i, 0, 0)),
        out_shape=jax.ShapeDtypeStruct((_NP // _BZ, 1, _BZ), jnp.float32),
    )(x, W1, b1r, W2)


def _stage_edges(ei_h, esd, wid):
    @pl.when(wid < _BIGT)
    def _():
        pltpu.sync_copy(ei_h.at[:, pl.ds(wid * _EPTA, _EPTA)], esd)

    @pl.when(wid >= _BIGT)
    def _():
        pltpu.sync_copy(
            ei_h.at[:, pl.ds(_BIGT * _EPTA + (wid - _BIGT) * _EPTB, _EPTB)],
            esd.at[:, pl.ds(0, _EPTB)])
        pad16 = jnp.full((_L,), _NP - 1, jnp.int32)
        def fill(i, _):
            esd[0, pl.ds(_EPTB + i * _L, _L)] = pad16
            esd[1, pl.ds(_EPTB + i * _L, _L)] = pad16
            return 0
        lax.fori_loop(0, (_EPTA - _EPTB) // _L, fill, 0)


def _deg_body(ei_h, deg_h, esd, agg, red, degc, shp, sem):
    wid = lax.axis_index("s")
    zero16 = jnp.zeros((_L,), jnp.float32)
    ones16 = jnp.ones((_L,), jnp.float32)
    _stage_edges(ei_h, esd, wid)

    @plsc.parallel_loop(0, _NP // _L, unroll=8)
    def _(i):
        agg[pl.ds(i * _L, _L)] = zero16

    @plsc.parallel_loop(0, _EPTA // _L, unroll=8)
    def _(i):
        dv = esd[1, pl.ds(i * _L, _L)]
        plsc.addupdate_scatter(agg, [dv], ones16)

    pltpu.sync_copy(agg, shp.at[wid])
    plsc.subcore_barrier()
    pltpu.sync_copy(shp.at[:, pl.ds(wid * _CHUNK, _CHUNK)], red)

    @plsc.parallel_loop(0, _CHUNK // _L, unroll=2)
    def _(i):
        ds = pl.ds(i * _L, _L)
        acc = red[0, ds]
        for t in range(1, _TILES):
            acc = acc + red[t, ds]
        degc[ds] = acc + 1.0
    pltpu.sync_copy(degc, deg_h.at[pl.ds(wid * _CHUNK, _CHUNK)])


def _deg_call(edge_index):
    mesh = plsc.VectorSubcoreMesh(core_axis_name="c", subcore_axis_name="s",
                                  num_cores=1, num_subcores=_TILES)
    return pl.kernel(
        _deg_body,
        out_type=jax.ShapeDtypeStruct((_NP,), jnp.float32),
        mesh=mesh,
        scratch_types=[
            pltpu.VMEM((2, _EPTA), jnp.int32),
            pltpu.VMEM((_NP,), jnp.float32),
            pltpu.VMEM((_TILES, _CHUNK), jnp.float32),
            pltpu.VMEM((_CHUNK,), jnp.float32),
            pltpu.VMEM_SHARED((_TILES, _NP), jnp.float32),
            pltpu.SemaphoreType.DMA,
        ],
        compiler_params=pltpu.CompilerParams(needs_layout_passes=False),
    )(edge_index)

def _rsqrt_newton(x):
    i = jnp.int32(0x5F3759DF) - (plsc.bitcast(x, jnp.int32) >> 1)
    r = plsc.bitcast(i, jnp.float32)
    for _ in range(3):
        r = r * (1.5 - 0.5 * x * r * r)
    return r


def _prop_body(ei_h, z0_h, deg_h, b2_h, out_h,
               esd, z0f, y, agg, red, y0c, dinvc, dsqc, outc, b2v,
               shp, shy, shz, sem):
    wid = lax.axis_index("s")
    zero16 = jnp.zeros((_L,), jnp.float32)
    last = _TILES - 1

    hb = pltpu.async_copy(b2_h, b2v, sem)
    hz = pltpu.async_copy(z0_h, z0f, sem)
    hd = pltpu.async_copy(deg_h.at[pl.ds(wid * _CHUNK, _CHUNK)], dsqc, sem)
    _stage_edges(ei_h, esd, wid)

    def _zero_agg():
        @plsc.parallel_loop(0, _NP // _L, unroll=8)
        def _(i):
            agg[pl.ds(i * _L, _L)] = zero16

    def _scatter_round():
        @plsc.parallel_loop(0, _EPTA // _L, unroll=16)
        def _(i):
            sv = esd[0, pl.ds(i * _L, _L)]
            dv = esd[1, pl.ds(i * _L, _L)]
            vals = plsc.load_gather(y, [sv])
            plsc.addupdate_scatter(agg, [dv], vals)

    def _publish_and_reduce(shp):
        pltpu.sync_copy(agg, shp.at[wid])
        plsc.subcore_barrier()
        pltpu.sync_copy(shp.at[:, pl.ds(wid * _CHUNK, _CHUNK)], red)

    def _reduce16(i):
        ds = pl.ds(i * _L, _L)
        acc = red[0, ds]
        for t in range(1, _TILES):
            acc = acc + red[t, ds]
        return ds, acc

    def run(shp, shy, shz):
        def _zoutc(i, _):
            outc[pl.ds(i * _L, _L)] = zero16
            return 0
        lax.fori_loop(0, _CHUNK // _L, _zoutc, 0)
        pltpu.sync_copy(outc, shz.at[pl.ds(wid * _CHUNK, _CHUNK)])
        _zero_agg()
        hb.wait()
        hz.wait()
        hd.wait()

        iota16 = lax.iota(jnp.int32, _L)

        @plsc.parallel_loop(0, _CHUNK // _L, unroll=2)
        def _(i):
            ds = pl.ds(i * _L, _L)
            deg = dsqc[ds]
            r = _rsqrt_newton(deg)
            dinvc[ds] = r * r
            dsqc[ds] = deg * r
            n = wid * _CHUNK + i * _L
            z = z0f[n >> 10, 0, pl.ds(n & (_BZ - 1), _L)]
            y0c[ds] = jnp.where(n + iota16 < _N, r * z, 0.0)
        pltpu.sync_copy(y0c, shy.at[pl.ds(wid * _CHUNK, _CHUNK)])
        plsc.subcore_barrier()
        pltpu.sync_copy(shy, y)

        def _round(k, _):
            _scatter_round()
            _publish_and_reduce(shp)
            zdma = pltpu.async_copy(shz, agg, sem)

            @plsc.parallel_loop(0, _CHUNK // _L, unroll=2)
            def _(i):
                ds, acc = _reduce16(i)
                yold = y[pl.ds(wid * _CHUNK + i * _L, _L)]
                outc[ds] = ((1.0 - _ALPHA) * dinvc[ds] * (acc + yold)
                            + _ALPHA * y0c[ds])
            pltpu.sync_copy(outc, shy.at[pl.ds(wid * _CHUNK, _CHUNK)])
            plsc.subcore_barrier()
            pltpu.sync_copy(shy, y)
            zdma.wait()
            return 0
        lax.fori_loop(0, _K - 1, _round, 0)

        _scatter_round()
        _publish_and_reduce(shp)
        bv = b2v[...]

        @plsc.parallel_loop(0, _CHUNK // _L, unroll=2)
        def _(i):
            ds, acc = _reduce16(i)
            yold = y[pl.ds(wid * _CHUNK + i * _L, _L)]
            ynew = ((1.0 - _ALPHA) * dinvc[ds] * (acc + yold)
                    + _ALPHA * y0c[ds])
            outc[ds] = dsqc[ds] * ynew + bv

        @pl.when(wid < last)
        def _():
            pltpu.sync_copy(outc, out_h.at[pl.ds(wid * _CHUNK, _CHUNK)])

        @pl.when(wid == last)
        def _():
            pltpu.sync_copy(outc.at[pl.ds(0, _ZLAST)],
                            out_h.at[pl.ds(last * _CHUNK, _ZLAST)])

    run(shp, shy, shz)


def _prop_call(edge_index, z0, deg, b2):
    mesh = plsc.VectorSubcoreMesh(core_axis_name="c", subcore_axis_name="s",
                                  num_cores=1, num_subcores=_TILES)
    return pl.kernel(
        _prop_body,
        out_type=jax.ShapeDtypeStruct((_N,), jnp.float32),
        mesh=mesh,
        scratch_types=[
            pltpu.VMEM((2, _EPTA), jnp.int32),
            pltpu.VMEM((_NP // _BZ, 1, _BZ), jnp.float32),
            pltpu.VMEM((_NP,), jnp.float32),
            pltpu.VMEM((_NP,), jnp.float32),
            pltpu.VMEM((_TILES, _CHUNK), jnp.float32),
            pltpu.VMEM((_CHUNK,), jnp.float32),
            pltpu.VMEM((_CHUNK,), jnp.float32),
            pltpu.VMEM((_CHUNK,), jnp.float32),
            pltpu.VMEM((_CHUNK,), jnp.float32),
            pltpu.VMEM((_L,), jnp.float32),
            pltpu.VMEM_SHARED((_TILES, _NP), jnp.float32),
            pltpu.VMEM_SHARED((_NP,), jnp.float32),
            pltpu.VMEM_SHARED((_NP,), jnp.float32),
            pltpu.SemaphoreType.DMA,
        ],
        compiler_params=pltpu.CompilerParams(needs_layout_passes=False),
    )(edge_index, z0, deg, jnp.broadcast_to(b2, (_L,)))


def kernel(x, edge_index, W1, b1, W2, b2):
    deg = _deg_call(edge_index)
    z0 = _mlp_call(x, W1, b1.reshape(1, _H), W2)
    outp = _prop_call(edge_index, z0, deg, b2)
    return outp.reshape(_N, 1)
